# baseline (speedup 1.0000x reference)
import jax
import jax.numpy as jnp
from jax.experimental import pallas as pl
from jax.experimental.pallas import tpu as pltpu

_Q = 2


def _rowdot_kernel(x_hbm, out_ref, *rest):
    bufs = rest[:_Q]
    sems = rest[_Q:]
    n = out_ref.shape[0]
    qcols = n // _Q
    copies = []
    for q in range(_Q):
        c = pltpu.make_async_copy(
            x_hbm.at[:, :, pl.ds(q * qcols, qcols)], bufs[q], sems[q]
        )
        c.start()
        copies.append(c)
    for q in range(_Q):
        copies[q].wait()
        out_ref[pl.ds(q * qcols, qcols)] = jnp.sum(
            bufs[q][0] * bufs[q][1], axis=0
        )


def kernel(inputs):
    n = inputs.shape[1]
    d = inputs.shape[2]
    t = jnp.swapaxes(inputs, 1, 2)
    qcols = n // _Q
    return pl.pallas_call(
        _rowdot_kernel,
        in_specs=[pl.BlockSpec(memory_space=pltpu.MemorySpace.HBM)],
        out_specs=pl.BlockSpec(memory_space=pltpu.MemorySpace.VMEM),
        out_shape=jax.ShapeDtypeStruct((n,), inputs.dtype),
        scratch_shapes=(
            [pltpu.VMEM((2, d, qcols), jnp.float32) for _ in range(_Q)]
            + [pltpu.SemaphoreType.DMA for _ in range(_Q)]
        ),
    )(t)


# speedup vs baseline: 1.0391x; 1.0391x over previous
import jax
import jax.numpy as jnp
from jax.experimental import pallas as pl
from jax.experimental.pallas import tpu as pltpu

_Q = 4


def _rowdot_kernel(x_hbm, out_ref, *rest):
    bufs = rest[:_Q]
    sems = rest[_Q:]
    n = out_ref.shape[0]
    qcols = n // _Q
    copies = []
    for q in range(_Q):
        c = pltpu.make_async_copy(
            x_hbm.at[:, :, pl.ds(q * qcols, qcols)], bufs[q], sems[q]
        )
        c.start()
        copies.append(c)
    for q in range(_Q):
        copies[q].wait()
        out_ref[pl.ds(q * qcols, qcols)] = jnp.sum(
            bufs[q][0] * bufs[q][1], axis=0
        )


def kernel(inputs):
    n = inputs.shape[1]
    d = inputs.shape[2]
    t = jnp.swapaxes(inputs, 1, 2)
    qcols = n // _Q
    return pl.pallas_call(
        _rowdot_kernel,
        in_specs=[pl.BlockSpec(memory_space=pltpu.MemorySpace.HBM)],
        out_specs=pl.BlockSpec(memory_space=pltpu.MemorySpace.VMEM),
        out_shape=jax.ShapeDtypeStruct((n,), inputs.dtype),
        scratch_shapes=(
            [pltpu.VMEM((2, d, qcols), jnp.float32) for _ in range(_Q)]
            + [pltpu.SemaphoreType.DMA for _ in range(_Q)]
        ),
    )(t)
